# baseline (speedup 1.0000x reference)
import jax
import jax.numpy as jnp
from jax.experimental import pallas as pl

N = 10000
HID = 256
L = 4
G = 64


def kernel(x, pos, edge_index, batch, e3_idx, emb_W, emb_b, msg_W1, msg_b1, msg_W2, msg_b2, upd_W1, upd_b1, upd_W2, upd_b2, e3_table, out_W1, out_b1, out_W2, out_b2, out_W3, out_b3):
    silu = jax.nn.silu
    row = edge_index[0]
    col = edge_index[1]
    ea = jnp.linalg.norm(pos[col] - pos[row], axis=-1, keepdims=True)
    h = x @ emb_W + emb_b
    for l in range(L):
        W1a = msg_W1[l][:HID]
        W1b = msg_W1[l][HID:2 * HID]
        w1c = msg_W1[l][2 * HID]
        P = h @ W1a + msg_b1[l]
        Q = h @ W1b
        m = silu(P[col] + Q[row] + ea * w1c)
        m = silu(m @ msg_W2[l] + msg_b2[l])
        aggr = jax.ops.segment_sum(m, col, num_segments=N)
        u = silu(h @ upd_W1[l][:HID] + aggr @ upd_W1[l][HID:] + upd_b1[l])
        u = u @ upd_W2[l] + upd_b2[l]
        h = h + u
    counts = jax.ops.segment_sum(jnp.ones((N,), jnp.float32), batch, num_segments=G)
    pooled = jax.ops.segment_sum(h, batch, num_segments=G) / counts[:, None]
    pooled = pooled / jnp.sqrt(counts)[:, None]
    e3 = e3_table[e3_idx]
    combined = jnp.concatenate([pooled, jnp.broadcast_to(e3, (G, HID))], axis=-1)
    o = silu(combined @ out_W1 + out_b1)
    o = silu(o @ out_W2 + out_b2)
    o = o @ out_W3 + out_b3
    return o


# speedup vs baseline: 1.2743x; 1.2743x over previous
import functools

import jax
import jax.numpy as jnp
from jax import lax
from jax.experimental import pallas as pl
from jax.experimental.pallas import tpu as pltpu
from jax.experimental.pallas import tpu_sc as plsc

N = 10000
NP = 10240
E = 320000
EP = 327680
D_IN = 128
H = 256
L = 4
NG = 64

NW = 32
EPW = EP // NW
C = 128
NCHUNK = EPW // C
EPT = EP // 16
RPT = NP // 16
BN = 1024
BE = 1024

_SC_MESH = plsc.VectorSubcoreMesh(core_axis_name="c", subcore_axis_name="s")


def _silu(v):
    return v / (1.0 + jnp.exp(-v))


def _make_gather(W):
    @functools.partial(
        pl.kernel,
        out_type=(jax.ShapeDtypeStruct((EP, W), jnp.float32),
                  jax.ShapeDtypeStruct((EP, W), jnp.float32)),
        mesh=_SC_MESH,
        scratch_types=[
            pltpu.VMEM((C,), jnp.int32),
            pltpu.VMEM((C,), jnp.int32),
            pltpu.VMEM((C, W), jnp.float32),
            pltpu.VMEM((C, W), jnp.float32),
            pltpu.SemaphoreType.DMA,
            pltpu.SemaphoreType.DMA,
        ],
    )
    def gather(h_hbm, row_hbm, col_hbm, xi_hbm, xj_hbm,
               ridx, cidx, bufi, bufj, semi, semj):
        wid = lax.axis_index("s") * 2 + lax.axis_index("c")
        base = wid * EPW

        def chunk(i, carry):
            off = base + i * C
            pltpu.sync_copy(row_hbm.at[pl.ds(off, C)], ridx)
            pltpu.sync_copy(col_hbm.at[pl.ds(off, C)], cidx)
            cpi = pltpu.async_copy(h_hbm.at[cidx], bufi, semi)
            cpj = pltpu.async_copy(h_hbm.at[ridx], bufj, semj)
            cpi.wait()
            pltpu.sync_copy(bufi, xi_hbm.at[pl.ds(off, C), :])
            cpj.wait()
            pltpu.sync_copy(bufj, xj_hbm.at[pl.ds(off, C), :])
            return carry

        lax.fori_loop(0, NCHUNK, chunk, 0)

    return gather


_sc_gather = _make_gather(H)
_sc_gather_pos = _make_gather(128)


@functools.partial(
    pl.kernel,
    out_type=jax.ShapeDtypeStruct((2, NP, 128), jnp.float32),
    mesh=_SC_MESH,
    scratch_types=[
        pltpu.VMEM((C,), jnp.int32),
        pltpu.VMEM((C, 128), jnp.float32),
        pltpu.VMEM_SHARED((NP, 128), jnp.float32),
    ],
)
def _sc_scatter(m_hbm, col_hbm, out_hbm, cidx, mbuf, accum):
    cc = lax.axis_index("c")
    ss = lax.axis_index("s")

    def zrow(r, carry):
        for j in range(8):
            mbuf[r, pl.ds(16 * j, 16)] = jnp.zeros((16,), jnp.float32)
        return carry

    lax.fori_loop(0, C, zrow, 0)

    def zcopy(k, carry):
        pltpu.sync_copy(mbuf, accum.at[pl.ds(ss * RPT + k * C, C), :])
        return carry

    lax.fori_loop(0, RPT // C, zcopy, 0)
    plsc.subcore_barrier()

    def chunk(i, carry):
        off = ss * EPT + i * C
        pltpu.sync_copy(col_hbm.at[pl.ds(off, C)], cidx)
        pltpu.sync_copy(m_hbm.at[cc, pl.ds(off, C), :], mbuf)
        pltpu.sync_copy(mbuf, accum.at[cidx], add=True)
        return carry

    lax.fori_loop(0, EPT // C, chunk, 0)
    plsc.subcore_barrier()

    def wb(k, carry):
        r0 = ss * RPT + k * C
        pltpu.sync_copy(accum.at[pl.ds(r0, C), :], mbuf)
        pltpu.sync_copy(mbuf, out_hbm.at[cc, pl.ds(r0, C), :])
        return carry

    lax.fori_loop(0, RPT // C, wb, 0)


def _tc_embed_body(x_ref, ew_ref, eb_ref, h_ref):
    h_ref[...] = jnp.dot(x_ref[...], ew_ref[...],
                         preferred_element_type=jnp.float32) + eb_ref[...]


def _tc_ea_body(pc_ref, pr_ref, ea_ref):
    d = pc_ref[...] - pr_ref[...]
    dx = d[:, 0:1]
    dy = d[:, 1:2]
    dz = d[:, 2:3]
    ss = (dx * dx + dy * dy) + dz * dz
    ea = jnp.sqrt(ss)
    lanei = lax.broadcasted_iota(jnp.int32, (BE, 128), 1)
    ea_ref[...] = jnp.where(lanei == 0, ea, 0.0)


def _tc_ea(pc, pr):
    return pl.pallas_call(
        _tc_ea_body,
        grid=(EP // BE,),
        in_specs=[
            pl.BlockSpec((BE, 128), lambda i: (i, 0)),
            pl.BlockSpec((BE, 128), lambda i: (i, 0)),
        ],
        out_specs=pl.BlockSpec((BE, 128), lambda i: (i, 0)),
        out_shape=jax.ShapeDtypeStruct((EP, 128), jnp.float32),
    )(pc, pr)


def _tc_edge_body(xi_ref, xj_ref, ea_ref, w1_ref, b1_ref, w2_ref, b2_ref,
                  m_ref):
    cat = jnp.concatenate([xi_ref[...], xj_ref[...], ea_ref[...]], axis=-1)
    m1 = jnp.dot(cat, w1_ref[...],
                 preferred_element_type=jnp.float32) + b1_ref[...]
    m = _silu(jnp.dot(_silu(m1), w2_ref[...],
                      preferred_element_type=jnp.float32) + b2_ref[...])
    m_ref[0, :, :] = m[:, :128]
    m_ref[1, :, :] = m[:, 128:]


def _tc_node_body(h_ref, ag_ref, u1_ref, ub1_ref, u2_ref, ub2_ref, hn_ref):
    h = h_ref[...]
    cat = jnp.concatenate([h, ag_ref[0, :, :], ag_ref[1, :, :]], axis=-1)
    u = _silu(jnp.dot(cat, u1_ref[...],
                      preferred_element_type=jnp.float32) + ub1_ref[...])
    hn_ref[...] = h + (jnp.dot(u, u2_ref[...],
                               preferred_element_type=jnp.float32)
                       + ub2_ref[...])


def _tc_readout_body(h_ref, b_ref, e3_ref, w1_ref, b1_ref,
                     w2_ref, b2_ref, w3_ref, b3_ref, out_ref, S, CNT):
    i = pl.program_id(0)

    @pl.when(i == 0)
    def _():
        S[...] = jnp.zeros_like(S)
        CNT[...] = jnp.zeros_like(CNT)

    bidx = b_ref[0, 0, :]
    oh = (bidx[:, None] == lax.broadcasted_iota(jnp.int32, (BN, NG), 1)
          ).astype(jnp.float32)
    dn = (((0,), (0,)), ((), ()))
    S[...] += lax.dot_general(oh, h_ref[...], dn,
                              preferred_element_type=jnp.float32,
                              precision=lax.Precision.HIGHEST)
    CNT[...] += lax.dot_general(oh, jnp.ones((BN, 128), jnp.float32), dn,
                                preferred_element_type=jnp.float32,
                                precision=lax.Precision.HIGHEST)

    @pl.when(i == NP // BN - 1)
    def _():
        cnt = CNT[:, 0:1]
        pooled = (S[...] / cnt) / jnp.sqrt(cnt)
        e3b = jnp.broadcast_to(e3_ref[...], (NG, H))
        comb = jnp.concatenate([pooled, e3b], axis=-1)
        o = _silu(jnp.dot(comb, w1_ref[...],
                          preferred_element_type=jnp.float32) + b1_ref[...])
        o = _silu(jnp.dot(o, w2_ref[...],
                          preferred_element_type=jnp.float32) + b2_ref[...])
        o = jnp.dot(o, w3_ref[...],
                    preferred_element_type=jnp.float32) + b3_ref[...]
        out_ref[...] = o


def _full(shape_nd):
    return pl.BlockSpec(shape_nd, lambda i: (0,) * len(shape_nd))


def _tc_embed(xp, emb_W, eb2):
    return pl.pallas_call(
        _tc_embed_body,
        grid=(NP // BN,),
        in_specs=[
            pl.BlockSpec((BN, D_IN), lambda i: (i, 0)),
            _full((D_IN, H)), _full((1, H)),
        ],
        out_specs=pl.BlockSpec((BN, H), lambda i: (i, 0)),
        out_shape=jax.ShapeDtypeStruct((NP, H), jnp.float32),
    )(xp, emb_W, eb2)


def _tc_edge(xi, xj, ea2, w1p, b12, w2, b22):
    return pl.pallas_call(
        _tc_edge_body,
        grid=(EP // BE,),
        in_specs=[
            pl.BlockSpec((BE, H), lambda i: (i, 0)),
            pl.BlockSpec((BE, H), lambda i: (i, 0)),
            pl.BlockSpec((BE, 128), lambda i: (i, 0)),
            _full((2 * H + 128, H)), _full((1, H)),
            _full((H, H)), _full((1, H)),
        ],
        out_specs=pl.BlockSpec((2, BE, 128), lambda i: (0, i, 0)),
        out_shape=jax.ShapeDtypeStruct((2, EP, 128), jnp.float32),
    )(xi, xj, ea2, w1p, b12, w2, b22)


def _tc_node(h, aggr, u1, ub12, u2, ub22):
    return pl.pallas_call(
        _tc_node_body,
        grid=(NP // BN,),
        in_specs=[
            pl.BlockSpec((BN, H), lambda i: (i, 0)),
            pl.BlockSpec((2, BN, 128), lambda i: (0, i, 0)),
            _full((2 * H, H)), _full((1, H)),
            _full((H, H)), _full((1, H)),
        ],
        out_specs=pl.BlockSpec((BN, H), lambda i: (i, 0)),
        out_shape=jax.ShapeDtypeStruct((NP, H), jnp.float32),
    )(h, aggr, u1, ub12, u2, ub22)


def _tc_readout(h, batch3, e32, ow1, ob12, ow2, ob22, ow3p, ob32):
    return pl.pallas_call(
        _tc_readout_body,
        grid=(NP // BN,),
        in_specs=[
            pl.BlockSpec((BN, H), lambda i: (i, 0)),
            pl.BlockSpec((1, 1, BN), lambda i: (i, 0, 0)),
            _full((1, H)),
            _full((2 * H, H)), _full((1, H)),
            _full((H, 128)), _full((1, 128)),
            _full((128, 128)), _full((1, 128)),
        ],
        out_specs=pl.BlockSpec((NG, 128), lambda i: (0, 0)),
        out_shape=jax.ShapeDtypeStruct((NG, 128), jnp.float32),
        scratch_shapes=[
            pltpu.VMEM((NG, H), jnp.float32),
            pltpu.VMEM((NG, 128), jnp.float32),
        ],
    )(h, batch3, e32, ow1, ob12, ow2, ob22, ow3p, ob32)


def kernel(x, pos, edge_index, batch, e3_idx, emb_W, emb_b, msg_W1, msg_b1,
           msg_W2, msg_b2, upd_W1, upd_b1, upd_W2, upd_b2, e3_table,
           out_W1, out_b1, out_W2, out_b2, out_W3, out_b3):
    row = edge_index[0].astype(jnp.int32)
    col = edge_index[1].astype(jnp.int32)
    pad_e = jnp.full((EP - E,), N, jnp.int32)
    rowp = jnp.concatenate([row, pad_e])
    colp = jnp.concatenate([col, pad_e])
    xp = jnp.pad(x, ((0, NP - N), (0, 0)))
    pos16 = jnp.pad(pos, ((0, NP - N), (0, 125)))
    batch3 = jnp.pad(batch.astype(jnp.int32), (0, NP - N),
                     constant_values=NG).reshape(NP // BN, 1, BN)

    r2 = lambda v: v.reshape(1, -1)
    w1p = jnp.pad(msg_W1, ((0, 0), (0, 127), (0, 0)))

    posc, posr = _sc_gather_pos(pos16, rowp, colp)
    ea2 = _tc_ea(posc, posr)
    h = _tc_embed(xp, emb_W, r2(emb_b))
    for l in range(L):
        xi, xj = _sc_gather(h, rowp, colp)
        m = _tc_edge(xi, xj, ea2, w1p[l], r2(msg_b1[l]), msg_W2[l],
                     r2(msg_b2[l]))
        aggr = _sc_scatter(m, colp)
        h = _tc_node(h, aggr, upd_W1[l], r2(upd_b1[l]), upd_W2[l],
                     r2(upd_b2[l]))

    e32 = e3_table[e3_idx].reshape(1, H)
    ow3p = jnp.pad(out_W3, ((0, 0), (0, 127)))
    ob32 = jnp.pad(out_b3, (0, 127)).reshape(1, 128)
    o = _tc_readout(h, batch3, e32, out_W1, r2(out_b1),
                    out_W2, r2(out_b2), ow3p, ob32)
    return o[:, :1]


# speedup vs baseline: 1.3068x; 1.0255x over previous
import functools

import jax
import jax.numpy as jnp
from jax import lax
from jax.experimental import pallas as pl
from jax.experimental.pallas import tpu as pltpu
from jax.experimental.pallas import tpu_sc as plsc

N = 10000
NP = 10240
E = 320000
EP = 327680
D_IN = 128
H = 256
L = 4
NG = 64

NW = 32
EPW = EP // NW
C = 128
NCHUNK = EPW // C
EPT = EP // 16
RPT = NP // 16
BN = 1024
BE = 1024
CG = 64
NCHUNKG = EPW // CG

_SC_MESH = plsc.VectorSubcoreMesh(core_axis_name="c", subcore_axis_name="s")


def _silu(v):
    return v / (1.0 + jnp.exp(-v))


def _make_gather(W):
    @functools.partial(
        pl.kernel,
        out_type=(jax.ShapeDtypeStruct((EP, W), jnp.float32),
                  jax.ShapeDtypeStruct((EP, W), jnp.float32)),
        mesh=_SC_MESH,
        scratch_types=[
            pltpu.VMEM((2, CG), jnp.int32),
            pltpu.VMEM((2, CG), jnp.int32),
            pltpu.VMEM((2, CG, W), jnp.float32),
            pltpu.VMEM((2, CG, W), jnp.float32),
            pltpu.SemaphoreType.DMA((2,)),
            pltpu.SemaphoreType.DMA((2,)),
            pltpu.SemaphoreType.DMA((2,)),
        ],
    )
    def gather(h_hbm, row_hbm, col_hbm, xi_hbm, xj_hbm,
               ridx, cidx, bufi, bufj, semi, semj, semw):
        wid = lax.axis_index("s") * 2 + lax.axis_index("c")
        base = wid * EPW

        def issue(i, b):
            off = base + i * CG
            pltpu.sync_copy(row_hbm.at[pl.ds(off, CG)], ridx.at[b])
            pltpu.sync_copy(col_hbm.at[pl.ds(off, CG)], cidx.at[b])
            pltpu.async_copy(h_hbm.at[cidx.at[b]], bufi.at[b], semi.at[b])
            pltpu.async_copy(h_hbm.at[ridx.at[b]], bufj.at[b], semj.at[b])

        def wait_wb(i, b):
            off = base + i * CG
            pltpu.make_async_copy(bufi.at[b], xi_hbm.at[pl.ds(off, CG), :],
                                  semw.at[b]).wait()
            pltpu.make_async_copy(bufj.at[b], xj_hbm.at[pl.ds(off, CG), :],
                                  semw.at[b]).wait()

        issue(0, 0)

        def pair(ip, carry):
            for b in range(2):
                i = ip * 2 + b
                nb = 1 - b

                @pl.when(i + 1 < NCHUNKG)
                def _():
                    @pl.when(i + 1 >= 2)
                    def _():
                        wait_wb(i - 1, nb)
                    issue(i + 1, nb)

                off = base + i * CG
                pltpu.make_async_copy(h_hbm.at[cidx.at[b]], bufi.at[b],
                                      semi.at[b]).wait()
                pltpu.async_copy(bufi.at[b], xi_hbm.at[pl.ds(off, CG), :],
                                 semw.at[b])
                pltpu.make_async_copy(h_hbm.at[ridx.at[b]], bufj.at[b],
                                      semj.at[b]).wait()
                pltpu.async_copy(bufj.at[b], xj_hbm.at[pl.ds(off, CG), :],
                                 semw.at[b])
            return carry

        lax.fori_loop(0, NCHUNKG // 2, pair, 0)
        wait_wb(NCHUNKG - 2, 0)
        wait_wb(NCHUNKG - 1, 1)

    return gather


_sc_gather = _make_gather(H)
_sc_gather_pos = _make_gather(128)


@functools.partial(
    pl.kernel,
    out_type=jax.ShapeDtypeStruct((2, NP, 128), jnp.float32),
    mesh=_SC_MESH,
    scratch_types=[
        pltpu.VMEM((C,), jnp.int32),
        pltpu.VMEM((C, 128), jnp.float32),
        pltpu.VMEM_SHARED((NP, 128), jnp.float32),
    ],
)
def _sc_scatter(m_hbm, col_hbm, out_hbm, cidx, mbuf, accum):
    cc = lax.axis_index("c")
    ss = lax.axis_index("s")

    def zrow(r, carry):
        for j in range(8):
            mbuf[r, pl.ds(16 * j, 16)] = jnp.zeros((16,), jnp.float32)
        return carry

    lax.fori_loop(0, C, zrow, 0)

    def zcopy(k, carry):
        pltpu.sync_copy(mbuf, accum.at[pl.ds(ss * RPT + k * C, C), :])
        return carry

    lax.fori_loop(0, RPT // C, zcopy, 0)
    plsc.subcore_barrier()

    def chunk(i, carry):
        off = ss * EPT + i * C
        pltpu.sync_copy(col_hbm.at[pl.ds(off, C)], cidx)
        pltpu.sync_copy(m_hbm.at[cc, pl.ds(off, C), :], mbuf)
        pltpu.sync_copy(mbuf, accum.at[cidx], add=True)
        return carry

    lax.fori_loop(0, EPT // C, chunk, 0)
    plsc.subcore_barrier()

    def wb(k, carry):
        r0 = ss * RPT + k * C
        pltpu.sync_copy(accum.at[pl.ds(r0, C), :], mbuf)
        pltpu.sync_copy(mbuf, out_hbm.at[cc, pl.ds(r0, C), :])
        return carry

    lax.fori_loop(0, RPT // C, wb, 0)


def _tc_embed_body(x_ref, ew_ref, eb_ref, h_ref):
    h_ref[...] = jnp.dot(x_ref[...], ew_ref[...],
                         preferred_element_type=jnp.float32) + eb_ref[...]


def _tc_edge_body(xi_ref, xj_ref, ea_ref, w1_ref, b1_ref, w2_ref, b2_ref,
                  m_ref):
    cat = jnp.concatenate([xi_ref[...], xj_ref[...], ea_ref[...]], axis=-1)
    m1 = jnp.dot(cat, w1_ref[...],
                 preferred_element_type=jnp.float32) + b1_ref[...]
    m = _silu(jnp.dot(_silu(m1), w2_ref[...],
                      preferred_element_type=jnp.float32) + b2_ref[...])
    m_ref[0, :, :] = m[:, :128]
    m_ref[1, :, :] = m[:, 128:]


def _tc_node_body(h_ref, ag_ref, u1_ref, ub1_ref, u2_ref, ub2_ref, hn_ref):
    h = h_ref[...]
    cat = jnp.concatenate([h, ag_ref[0, :, :], ag_ref[1, :, :]], axis=-1)
    u = _silu(jnp.dot(cat, u1_ref[...],
                      preferred_element_type=jnp.float32) + ub1_ref[...])
    hn_ref[...] = h + (jnp.dot(u, u2_ref[...],
                               preferred_element_type=jnp.float32)
                       + ub2_ref[...])


def _tc_readout_body(h_ref, b_ref, e3_ref, w1_ref, b1_ref,
                     w2_ref, b2_ref, w3_ref, b3_ref, out_ref, S, CNT):
    i = pl.program_id(0)

    @pl.when(i == 0)
    def _():
        S[...] = jnp.zeros_like(S)
        CNT[...] = jnp.zeros_like(CNT)

    bidx = b_ref[0, 0, :]
    oh = (bidx[:, None] == lax.broadcasted_iota(jnp.int32, (BN, NG), 1)
          ).astype(jnp.float32)
    dn = (((0,), (0,)), ((), ()))
    S[...] += lax.dot_general(oh, h_ref[...], dn,
                              preferred_element_type=jnp.float32,
                              precision=lax.Precision.HIGHEST)
    CNT[...] += lax.dot_general(oh, jnp.ones((BN, 128), jnp.float32), dn,
                                preferred_element_type=jnp.float32,
                                precision=lax.Precision.HIGHEST)

    @pl.when(i == NP // BN - 1)
    def _():
        cnt = CNT[:, 0:1]
        pooled = (S[...] / cnt) / jnp.sqrt(cnt)
        e3b = jnp.broadcast_to(e3_ref[...], (NG, H))
        comb = jnp.concatenate([pooled, e3b], axis=-1)
        o = _silu(jnp.dot(comb, w1_ref[...],
                          preferred_element_type=jnp.float32) + b1_ref[...])
        o = _silu(jnp.dot(o, w2_ref[...],
                          preferred_element_type=jnp.float32) + b2_ref[...])
        o = jnp.dot(o, w3_ref[...],
                    preferred_element_type=jnp.float32) + b3_ref[...]
        out_ref[...] = o


def _full(shape_nd):
    return pl.BlockSpec(shape_nd, lambda i: (0,) * len(shape_nd))


def _tc_embed(xp, emb_W, eb2):
    return pl.pallas_call(
        _tc_embed_body,
        grid=(NP // BN,),
        in_specs=[
            pl.BlockSpec((BN, D_IN), lambda i: (i, 0)),
            _full((D_IN, H)), _full((1, H)),
        ],
        out_specs=pl.BlockSpec((BN, H), lambda i: (i, 0)),
        out_shape=jax.ShapeDtypeStruct((NP, H), jnp.float32),
    )(xp, emb_W, eb2)


def _tc_edge(xi, xj, ea2, w1p, b12, w2, b22):
    return pl.pallas_call(
        _tc_edge_body,
        grid=(EP // BE,),
        in_specs=[
            pl.BlockSpec((BE, H), lambda i: (i, 0)),
            pl.BlockSpec((BE, H), lambda i: (i, 0)),
            pl.BlockSpec((BE, 128), lambda i: (i, 0)),
            _full((2 * H + 128, H)), _full((1, H)),
            _full((H, H)), _full((1, H)),
        ],
        out_specs=pl.BlockSpec((2, BE, 128), lambda i: (0, i, 0)),
        out_shape=jax.ShapeDtypeStruct((2, EP, 128), jnp.float32),
    )(xi, xj, ea2, w1p, b12, w2, b22)


def _tc_node(h, aggr, u1, ub12, u2, ub22):
    return pl.pallas_call(
        _tc_node_body,
        grid=(NP // BN,),
        in_specs=[
            pl.BlockSpec((BN, H), lambda i: (i, 0)),
            pl.BlockSpec((2, BN, 128), lambda i: (0, i, 0)),
            _full((2 * H, H)), _full((1, H)),
            _full((H, H)), _full((1, H)),
        ],
        out_specs=pl.BlockSpec((BN, H), lambda i: (i, 0)),
        out_shape=jax.ShapeDtypeStruct((NP, H), jnp.float32),
    )(h, aggr, u1, ub12, u2, ub22)


def _tc_readout(h, batch3, e32, ow1, ob12, ow2, ob22, ow3p, ob32):
    return pl.pallas_call(
        _tc_readout_body,
        grid=(NP // BN,),
        in_specs=[
            pl.BlockSpec((BN, H), lambda i: (i, 0)),
            pl.BlockSpec((1, 1, BN), lambda i: (i, 0, 0)),
            _full((1, H)),
            _full((2 * H, H)), _full((1, H)),
            _full((H, 128)), _full((1, 128)),
            _full((128, 128)), _full((1, 128)),
        ],
        out_specs=pl.BlockSpec((NG, 128), lambda i: (0, 0)),
        out_shape=jax.ShapeDtypeStruct((NG, 128), jnp.float32),
        scratch_shapes=[
            pltpu.VMEM((NG, H), jnp.float32),
            pltpu.VMEM((NG, 128), jnp.float32),
        ],
    )(h, batch3, e32, ow1, ob12, ow2, ob22, ow3p, ob32)


def kernel(x, pos, edge_index, batch, e3_idx, emb_W, emb_b, msg_W1, msg_b1,
           msg_W2, msg_b2, upd_W1, upd_b1, upd_W2, upd_b2, e3_table,
           out_W1, out_b1, out_W2, out_b2, out_W3, out_b3):
    row = edge_index[0].astype(jnp.int32)
    col = edge_index[1].astype(jnp.int32)
    pad_e = jnp.full((EP - E,), N, jnp.int32)
    rowp = jnp.concatenate([row, pad_e])
    colp = jnp.concatenate([col, pad_e])
    xp = jnp.pad(x, ((0, NP - N), (0, 0)))
    pos16 = jnp.pad(pos, ((0, NP - N), (0, 125)))
    batch3 = jnp.pad(batch.astype(jnp.int32), (0, NP - N),
                     constant_values=NG).reshape(NP // BN, 1, BN)

    r2 = lambda v: v.reshape(1, -1)
    w1p = jnp.pad(msg_W1, ((0, 0), (0, 127), (0, 0)))

    posc, posr = _sc_gather_pos(pos16, rowp, colp)
    ea = jnp.linalg.norm(posc[:, :3] - posr[:, :3], axis=-1)
    ea2 = jnp.pad(ea[:, None], ((0, 0), (0, 127)))
    h = _tc_embed(xp, emb_W, r2(emb_b))
    for l in range(L):
        xi, xj = _sc_gather(h, rowp, colp)
        m = _tc_edge(xi, xj, ea2, w1p[l], r2(msg_b1[l]), msg_W2[l],
                     r2(msg_b2[l]))
        aggr = _sc_scatter(m, colp)
        h = _tc_node(h, aggr, upd_W1[l], r2(upd_b1[l]), upd_W2[l],
                     r2(upd_b2[l]))

    e32 = e3_table[e3_idx].reshape(1, H)
    ow3p = jnp.pad(out_W3, ((0, 0), (0, 127)))
    ob32 = jnp.pad(out_b3, (0, 127)).reshape(1, 128)
    o = _tc_readout(h, batch3, e32, out_W1, r2(out_b1),
                    out_W2, r2(out_b2), ow3p, ob32)
    return o[:, :1]
